# baseline (speedup 1.0000x reference)
import functools

import jax
import jax.numpy as jnp
from jax import lax
from jax.experimental import pallas as pl
from jax.experimental.pallas import tpu as pltpu
from jax.experimental.pallas import tpu_sc as plsc

N_CAT = 26
N_NUM = 13
VOCAB = 100000
DIM = 32
B = 16384
NC, NS = 2, 16
NW = NC * NS
SL = 128
GRP = 4
GROUP_ROWS = GRP * SL

VBLK = 512
NBLK = 196
LAST_BASE = VOCAB - VBLK
BPG = 28
NGSTEP = NBLK // BPG

FGROUPS = ((0, 8), (8, 16), (16, 24), (24, 26))

XT_RB, XT_S, XT_L = B // 8, 8, 128


def _relayout_tables(t2g, nf):
    SPAN = BPG * VBLK

    def body(t2_ref, out_ref):
        i = pl.program_id(1)

        @pl.when(i < NGSTEP - 1)
        def _():
            for k in range(BPG):
                x = t2_ref[0, :, pl.ds(k * VBLK, VBLK)]
                s = jnp.concatenate([x[:, 0:128], x[:, 128:256],
                                     x[:, 256:384], x[:, 384:512]], axis=0)
                out_ref[pl.ds(k * 128, 128), :] = s.T

        @pl.when(i == NGSTEP - 1)
        def _():
            for k in range(BPG):
                start = min((NGSTEP - 1) * SPAN + k * VBLK,
                            LAST_BASE) - (NGSTEP - 1) * SPAN
                x = t2_ref[0, :, pl.ds(start, VBLK)]
                s = jnp.concatenate([x[:, 0:128], x[:, 128:256],
                                     x[:, 256:384], x[:, 384:512]], axis=0)
                out_ref[pl.ds(k * 128, 128), :] = s.T

    return pl.pallas_call(
        body,
        grid=(nf, NGSTEP),
        compiler_params=pltpu.CompilerParams(
            dimension_semantics=("parallel", "parallel")),
        in_specs=[pl.BlockSpec((1, DIM, SPAN), lambda f, i: (f, 0, i))],
        out_specs=pl.BlockSpec((BPG * 128, 128), lambda f, i: (f * NGSTEP + i, 0)),
        out_shape=jax.ShapeDtypeStruct((nf * NBLK * 128, 128), jnp.float32),
    )(t2g)


def _sc_gather_scatter(flat_tables, idx3, scat3, nslice, out_chunks):
    mesh = plsc.VectorSubcoreMesh(core_axis_name="c", subcore_axis_name="s")
    ngrp = nslice // GRP

    @functools.partial(
        pl.kernel,
        mesh=mesh,
        compiler_params=pltpu.CompilerParams(use_tc_tiling_on_sc=False),
        out_type=jax.ShapeDtypeStruct((out_chunks, DIM), jnp.float32),
        scratch_types=[
            pltpu.VMEM((nslice, SL), jnp.int32),
            pltpu.VMEM((nslice, SL), jnp.int32),
            pltpu.VMEM((GROUP_ROWS, DIM), jnp.float32),
            pltpu.VMEM((GROUP_ROWS, DIM), jnp.float32),
            pltpu.SemaphoreType.DMA,
            pltpu.SemaphoreType.DMA,
            pltpu.SemaphoreType.DMA,
            pltpu.SemaphoreType.DMA,
        ],
    )
    def k(tab_hbm, idx_hbm, scat_hbm, out_hbm, idx_v, scat_v,
          buf0, buf1, gsem0, gsem1, wsem0, wsem1):
        wid = lax.axis_index("s") * NC + lax.axis_index("c")
        pltpu.sync_copy(idx_hbm.at[wid], idx_v)
        pltpu.sync_copy(scat_hbm.at[wid], scat_v)

        def fire_gather(g, buf, sem):
            for j in range(GRP):
                pltpu.async_copy(
                    tab_hbm.at[idx_v.at[g * GRP + j]],
                    buf.at[pl.ds(j * SL, SL)], sem)

        def drain(buf, sem, n=GRP):
            for j in range(n):
                pltpu.make_async_copy(
                    tab_hbm.at[idx_v.at[0]], buf.at[pl.ds(j * SL, SL)], sem
                ).wait()

        def fire_scatter(g, buf, sem):
            for j in range(GRP):
                pltpu.async_copy(
                    buf.at[pl.ds(j * SL, SL)],
                    out_hbm.at[scat_v.at[g * GRP + j]], sem)

        fire_gather(0, buf0, gsem0)

        def body(g, carry):
            def phase(buf, gsem, wsem, obuf, ogsem):
                drain(buf, gsem)
                @pl.when(g + 1 < ngrp)
                def _():
                    fire_gather(g + 1, obuf, ogsem)
                fire_scatter(g, buf, wsem)
                drain(buf, wsem)

            @pl.when(g % 2 == 0)
            def _():
                phase(buf0, gsem0, wsem0, buf1, gsem1)

            @pl.when(g % 2 == 1)
            def _():
                phase(buf1, gsem1, wsem1, buf0, gsem0)

            return carry

        lax.fori_loop(0, ngrp, body, 0)

    return k(flat_tables, idx3, scat3)


def _mlp(slabs, numeric, W1, b1r, W2, b2r):
    BK = 1024
    BKH = BK // 8
    jts = [(f1 - f0) // 4 if (f1 - f0) % 4 == 0 else (f1 - f0 + 3) // 4
           for f0, f1 in FGROUPS]

    def body(s0_ref, s1_ref, s2_ref, s3_ref, num_ref,
             w1_ref, b1_ref, w2_ref, b2_ref, out_ref):
        parts = []
        for ref, jt in zip((s0_ref, s1_ref, s2_ref, s3_ref), jts):
            for jj in range(jt):
                p = ref[:, jj, :, :].reshape(BK, 128)
                parts.append(p)
        parts[-1] = parts[-1][:, :64]
        parts.append(num_ref[...])
        x = jnp.concatenate(parts, axis=1)
        h = jnp.dot(x, w1_ref[...], preferred_element_type=jnp.float32)
        h = jnp.maximum(h + b1_ref[...], 0.0)
        o = jnp.dot(h, w2_ref[...], preferred_element_type=jnp.float32) + b2_ref[0, 0]
        out_ref[...] = 1.0 / (1.0 + jnp.exp(-o))

    slab_specs = [
        pl.BlockSpec((BKH, jt, XT_S, XT_L), lambda i: (i, 0, 0, 0))
        for jt in jts
    ]
    return pl.pallas_call(
        body,
        grid=(B // BK,),
        in_specs=slab_specs + [
            pl.BlockSpec((BK, N_NUM), lambda i: (i, 0)),
            pl.BlockSpec((N_CAT * DIM + N_NUM, 128), lambda i: (0, 0)),
            pl.BlockSpec((1, 128), lambda i: (0, 0)),
            pl.BlockSpec((128, 1), lambda i: (0, 0)),
            pl.BlockSpec((1, 1), lambda i: (0, 0)),
        ],
        out_specs=pl.BlockSpec((BK, 1), lambda i: (i, 0)),
        out_shape=jax.ShapeDtypeStruct((B, 1), jnp.float32),
    )(*slabs, numeric, W1, b1r, W2, b2r)


def kernel(inputs, tables, W1, b1, W2, b2):
    idx = inputs[:, :N_CAT].astype(jnp.int32)
    t2 = jnp.swapaxes(tables, 1, 2)
    bb = jnp.arange(B, dtype=jnp.int32)[:, None]

    slabs = []
    for f0, f1 in FGROUPS:
        nf = f1 - f0
        jt = (nf + 3) // 4
        nslice = B * nf // (NW * SL)

        idx_g = idx[:, f0:f1]
        fl = jnp.arange(nf, dtype=jnp.int32)[None, :]
        g = jnp.minimum(idx_g // VBLK, NBLK - 1)
        r = idx_g - jnp.minimum(g * VBLK, LAST_BASE)
        flat_idx = ((fl * NBLK + g) * 128 + r % 128) * 4 + r // 128
        idx3 = flat_idx.reshape(NW, nslice, SL)

        ii = jnp.arange(f0, f1, dtype=jnp.int32)[None, :]
        scat = ((bb // 8) * (jt * 32) + (ii // 4 - f0 // 4) * 32
                + (bb % 8) * 4 + (ii % 4))
        scat3 = scat.reshape(NW, nslice, SL)

        tab_lines = _relayout_tables(t2[f0:f1], nf)
        flat_tab = tab_lines.reshape(nf * NBLK * VBLK, DIM)

        out_chunks = XT_RB * jt * XT_S * XT_L // DIM
        xflat = _sc_gather_scatter(flat_tab, idx3, scat3, nslice, out_chunks)
        slabs.append(xflat.reshape(XT_RB, jt, XT_S, XT_L))

    numeric = inputs[:, N_CAT:]
    return _mlp(slabs, numeric, W1, b1.reshape(1, 128), W2, b2.reshape(1, 1))


# speedup vs baseline: 1.7727x; 1.7727x over previous
import functools

import jax
import jax.numpy as jnp
from jax import lax
from jax.experimental import pallas as pl
from jax.experimental.pallas import tpu as pltpu
from jax.experimental.pallas import tpu_sc as plsc

N_CAT = 26
N_NUM = 13
VOCAB = 100000
DIM = 32
B = 16384
ROWS = B * N_CAT
NC, NS = 2, 16
NW = NC * NS
ROWS_PER_W = ROWS // NW
SL = 128
NSLICE = ROWS_PER_W // SL
GRP = 4
GROUP_ROWS = GRP * SL
NGRP = NSLICE // GRP

VBLK = 512
NBLK = 196
LAST_BASE = VOCAB - VBLK
BPG = 49
NGSTEP = NBLK // BPG
LINES = N_CAT * NBLK * 128
TROWS = LINES * 4

XT_RB, XT_J, XT_S, XT_L = B // 8, 7, 8, 128
XCHUNKS = XT_RB * XT_J * XT_S * XT_L // DIM


def _relayout_tables(t2):
    SPAN = BPG * VBLK

    def body(t2_ref, out_ref):
        i = pl.program_id(1)

        @pl.when(i < NGSTEP - 1)
        def _():
            for k in range(BPG):
                x = t2_ref[0, :, pl.ds(k * VBLK, VBLK)]
                s = jnp.concatenate([x[:, 0:128], x[:, 128:256],
                                     x[:, 256:384], x[:, 384:512]], axis=0)
                out_ref[pl.ds(k * 128, 128), :] = s.T

        @pl.when(i == NGSTEP - 1)
        def _():
            for k in range(BPG):
                start = min((NGSTEP - 1) * SPAN + k * VBLK,
                            LAST_BASE) - (NGSTEP - 1) * SPAN
                x = t2_ref[0, :, pl.ds(start, VBLK)]
                s = jnp.concatenate([x[:, 0:128], x[:, 128:256],
                                     x[:, 256:384], x[:, 384:512]], axis=0)
                out_ref[pl.ds(k * 128, 128), :] = s.T

    return pl.pallas_call(
        body,
        grid=(N_CAT, NGSTEP),
        in_specs=[pl.BlockSpec((1, DIM, SPAN), lambda f, i: (f, 0, i))],
        out_specs=pl.BlockSpec((BPG * 128, 128), lambda f, i: (f * NGSTEP + i, 0)),
        out_shape=jax.ShapeDtypeStruct((LINES, 128), jnp.float32),
    )(t2)


def _sc_gather_scatter(flat_tables, idx3, scat3):
    mesh = plsc.VectorSubcoreMesh(core_axis_name="c", subcore_axis_name="s")

    @functools.partial(
        pl.kernel,
        mesh=mesh,
        compiler_params=pltpu.CompilerParams(use_tc_tiling_on_sc=False),
        out_type=jax.ShapeDtypeStruct((XCHUNKS, DIM), jnp.float32),
        scratch_types=[
            pltpu.VMEM((NSLICE, SL), jnp.int32),
            pltpu.VMEM((NSLICE, SL), jnp.int32),
            pltpu.VMEM((GROUP_ROWS, DIM), jnp.float32),
            pltpu.VMEM((GROUP_ROWS, DIM), jnp.float32),
            pltpu.SemaphoreType.DMA,
            pltpu.SemaphoreType.DMA,
            pltpu.SemaphoreType.DMA,
            pltpu.SemaphoreType.DMA,
        ],
    )
    def k(tab_hbm, idx_hbm, scat_hbm, out_hbm, idx_v, scat_v,
          buf0, buf1, gsem0, gsem1, wsem0, wsem1):
        wid = lax.axis_index("s") * NC + lax.axis_index("c")
        pltpu.sync_copy(idx_hbm.at[wid], idx_v)
        pltpu.sync_copy(scat_hbm.at[wid], scat_v)

        def fire_gather(g, buf, sem):
            for j in range(GRP):
                pltpu.async_copy(
                    tab_hbm.at[idx_v.at[g * GRP + j]],
                    buf.at[pl.ds(j * SL, SL)], sem)

        def drain(buf, sem, n=GRP):
            for j in range(n):
                pltpu.make_async_copy(
                    tab_hbm.at[idx_v.at[0]], buf.at[pl.ds(j * SL, SL)], sem
                ).wait()

        def fire_scatter(g, buf, sem):
            for j in range(GRP):
                pltpu.async_copy(
                    buf.at[pl.ds(j * SL, SL)],
                    out_hbm.at[scat_v.at[g * GRP + j]], sem)

        fire_gather(0, buf0, gsem0)

        def body(g, carry):
            def phase(buf, gsem, wsem, obuf, ogsem):
                drain(buf, gsem)
                @pl.when(g + 1 < NGRP)
                def _():
                    fire_gather(g + 1, obuf, ogsem)
                fire_scatter(g, buf, wsem)
                drain(buf, wsem)

            @pl.when(g % 2 == 0)
            def _():
                phase(buf0, gsem0, wsem0, buf1, gsem1)

            @pl.when(g % 2 == 1)
            def _():
                phase(buf1, gsem1, wsem1, buf0, gsem0)

            return carry

        lax.fori_loop(0, NGRP, body, 0)

    return k(flat_tables, idx3, scat3)


def _mlp(x4, numeric, W1, b1r, W2, b2r):
    BK = 1024
    BKH = BK // 8

    def body(x4_ref, num_ref, w1_ref, b1_ref, w2_ref, b2_ref, out_ref):
        parts = [x4_ref[:, j, :, :].reshape(BK, 128) for j in range(XT_J - 1)]
        parts.append(x4_ref[:, XT_J - 1, :, :].reshape(BK, 128)[:, :64])
        parts.append(num_ref[...])
        x = jnp.concatenate(parts, axis=1)
        h = jnp.dot(x, w1_ref[...], preferred_element_type=jnp.float32)
        h = jnp.maximum(h + b1_ref[...], 0.0)
        o = jnp.dot(h, w2_ref[...], preferred_element_type=jnp.float32) + b2_ref[0, 0]
        out_ref[...] = 1.0 / (1.0 + jnp.exp(-o))

    return pl.pallas_call(
        body,
        grid=(B // BK,),
        in_specs=[
            pl.BlockSpec((BKH, XT_J, XT_S, XT_L), lambda i: (i, 0, 0, 0)),
            pl.BlockSpec((BK, N_NUM), lambda i: (i, 0)),
            pl.BlockSpec((N_CAT * DIM + N_NUM, 128), lambda i: (0, 0)),
            pl.BlockSpec((1, 128), lambda i: (0, 0)),
            pl.BlockSpec((128, 1), lambda i: (0, 0)),
            pl.BlockSpec((1, 1), lambda i: (0, 0)),
        ],
        out_specs=pl.BlockSpec((BK, 1), lambda i: (i, 0)),
        out_shape=jax.ShapeDtypeStruct((B, 1), jnp.float32),
    )(x4, numeric, W1, b1r, W2, b2r)


def kernel(inputs, tables, W1, b1, W2, b2):
    idx = inputs[:, :N_CAT].astype(jnp.int32)
    ff = jnp.arange(N_CAT, dtype=jnp.int32)[None, :]
    g = jnp.minimum(idx // VBLK, NBLK - 1)
    r = idx - jnp.minimum(g * VBLK, LAST_BASE)
    flat_idx = ((ff * NBLK + g) * 128 + r % 128) * 4 + r // 128
    idx3 = flat_idx.reshape(NW, NSLICE, SL)

    bb = jnp.arange(B, dtype=jnp.int32)[:, None]
    ii = jnp.arange(N_CAT, dtype=jnp.int32)[None, :]
    scat = ((bb // 8) * (XT_J * 32) + (ii // 4) * 32 + (bb % 8) * 4 + (ii % 4))
    scat3 = scat.reshape(NW, NSLICE, SL)

    t2 = jnp.swapaxes(tables, 1, 2)
    tab_lines = _relayout_tables(t2)
    flat_tables = tab_lines.reshape(TROWS, DIM)

    xflat = _sc_gather_scatter(flat_tables, idx3, scat3)
    x4 = xflat.reshape(XT_RB, XT_J, XT_S, XT_L)

    numeric = inputs[:, N_CAT:]
    return _mlp(x4, numeric, W1, b1.reshape(1, 128), W2, b2.reshape(1, 1))


# speedup vs baseline: 1.8336x; 1.0343x over previous
import functools

import jax
import jax.numpy as jnp
from jax import lax
from jax.experimental import pallas as pl
from jax.experimental.pallas import tpu as pltpu
from jax.experimental.pallas import tpu_sc as plsc

N_CAT = 26
N_NUM = 13
VOCAB = 100000
DIM = 32
B = 16384
ROWS = B * N_CAT
NC, NS = 2, 16
NW = NC * NS
ROWS_PER_W = ROWS // NW
SL = 128
NSLICE = ROWS_PER_W // SL
GRP = 4
GROUP_ROWS = GRP * SL
NGRP = NSLICE // GRP

VBLK = 512
NBLK = 196
LAST_BASE = VOCAB - VBLK
BPG = 98
NGSTEP = NBLK // BPG
LINES = N_CAT * NBLK * 128
TROWS = LINES * 4

XT_RB, XT_J, XT_S, XT_L = B // 8, 7, 8, 128
XCHUNKS = XT_RB * XT_J * XT_S * XT_L // DIM


def _relayout_tables(t2):
    SPAN = BPG * VBLK

    def body(t2_ref, out_ref):
        i = pl.program_id(1)

        @pl.when(i < NGSTEP - 1)
        def _():
            for k in range(BPG):
                x = t2_ref[0, :, pl.ds(k * VBLK, VBLK)]
                s = jnp.concatenate([x[:, 0:128], x[:, 128:256],
                                     x[:, 256:384], x[:, 384:512]], axis=0)
                out_ref[pl.ds(k * 128, 128), :] = s.T

        @pl.when(i == NGSTEP - 1)
        def _():
            for k in range(BPG):
                start = min((NGSTEP - 1) * SPAN + k * VBLK,
                            LAST_BASE) - (NGSTEP - 1) * SPAN
                x = t2_ref[0, :, pl.ds(start, VBLK)]
                s = jnp.concatenate([x[:, 0:128], x[:, 128:256],
                                     x[:, 256:384], x[:, 384:512]], axis=0)
                out_ref[pl.ds(k * 128, 128), :] = s.T

    return pl.pallas_call(
        body,
        grid=(N_CAT, NGSTEP),
        in_specs=[pl.BlockSpec((1, DIM, SPAN), lambda f, i: (f, 0, i))],
        out_specs=pl.BlockSpec((BPG * 128, 128), lambda f, i: (f * NGSTEP + i, 0)),
        out_shape=jax.ShapeDtypeStruct((LINES, 128), jnp.float32),
    )(t2)


def _sc_gather_scatter(flat_tables, idx3, scat3):
    mesh = plsc.VectorSubcoreMesh(core_axis_name="c", subcore_axis_name="s")

    @functools.partial(
        pl.kernel,
        mesh=mesh,
        compiler_params=pltpu.CompilerParams(use_tc_tiling_on_sc=False),
        out_type=jax.ShapeDtypeStruct((XCHUNKS, DIM), jnp.float32),
        scratch_types=[
            pltpu.VMEM((NSLICE, SL), jnp.int32),
            pltpu.VMEM((NSLICE, SL), jnp.int32),
            pltpu.VMEM((GROUP_ROWS, DIM), jnp.float32),
            pltpu.VMEM((GROUP_ROWS, DIM), jnp.float32),
            pltpu.SemaphoreType.DMA,
            pltpu.SemaphoreType.DMA,
            pltpu.SemaphoreType.DMA,
            pltpu.SemaphoreType.DMA,
        ],
    )
    def k(tab_hbm, idx_hbm, scat_hbm, out_hbm, idx_v, scat_v,
          buf0, buf1, gsem0, gsem1, wsem0, wsem1):
        wid = lax.axis_index("s") * NC + lax.axis_index("c")
        pltpu.sync_copy(idx_hbm.at[wid], idx_v)
        pltpu.sync_copy(scat_hbm.at[wid], scat_v)

        def fire_gather(g, buf, sem):
            for j in range(GRP):
                pltpu.async_copy(
                    tab_hbm.at[idx_v.at[g * GRP + j]],
                    buf.at[pl.ds(j * SL, SL)], sem)

        def drain(buf, sem, n=GRP):
            for j in range(n):
                pltpu.make_async_copy(
                    tab_hbm.at[idx_v.at[0]], buf.at[pl.ds(j * SL, SL)], sem
                ).wait()

        def fire_scatter(g, buf, sem):
            for j in range(GRP):
                pltpu.async_copy(
                    buf.at[pl.ds(j * SL, SL)],
                    out_hbm.at[scat_v.at[g * GRP + j]], sem)

        fire_gather(0, buf0, gsem0)

        def body(g, carry):
            def phase(buf, gsem, wsem, obuf, ogsem):
                drain(buf, gsem)
                @pl.when(g + 1 < NGRP)
                def _():
                    fire_gather(g + 1, obuf, ogsem)
                fire_scatter(g, buf, wsem)
                drain(buf, wsem)

            @pl.when(g % 2 == 0)
            def _():
                phase(buf0, gsem0, wsem0, buf1, gsem1)

            @pl.when(g % 2 == 1)
            def _():
                phase(buf1, gsem1, wsem1, buf0, gsem0)

            return carry

        lax.fori_loop(0, NGRP, body, 0)

    return k(flat_tables, idx3, scat3)


def _mlp(x4, numeric, W1, b1r, W2, b2r):
    BK = 1024
    BKH = BK // 8

    def body(x4_ref, num_ref, w1_ref, b1_ref, w2_ref, b2_ref, out_ref):
        parts = [x4_ref[:, j, :, :].reshape(BK, 128) for j in range(XT_J - 1)]
        parts.append(x4_ref[:, XT_J - 1, :, :].reshape(BK, 128)[:, :64])
        parts.append(num_ref[...])
        x = jnp.concatenate(parts, axis=1)
        h = jnp.dot(x, w1_ref[...], preferred_element_type=jnp.float32)
        h = jnp.maximum(h + b1_ref[...], 0.0)
        o = jnp.dot(h, w2_ref[...], preferred_element_type=jnp.float32) + b2_ref[0, 0]
        out_ref[...] = 1.0 / (1.0 + jnp.exp(-o))

    return pl.pallas_call(
        body,
        grid=(B // BK,),
        in_specs=[
            pl.BlockSpec((BKH, XT_J, XT_S, XT_L), lambda i: (i, 0, 0, 0)),
            pl.BlockSpec((BK, N_NUM), lambda i: (i, 0)),
            pl.BlockSpec((N_CAT * DIM + N_NUM, 128), lambda i: (0, 0)),
            pl.BlockSpec((1, 128), lambda i: (0, 0)),
            pl.BlockSpec((128, 1), lambda i: (0, 0)),
            pl.BlockSpec((1, 1), lambda i: (0, 0)),
        ],
        out_specs=pl.BlockSpec((BK, 1), lambda i: (i, 0)),
        out_shape=jax.ShapeDtypeStruct((B, 1), jnp.float32),
    )(x4, numeric, W1, b1r, W2, b2r)


def kernel(inputs, tables, W1, b1, W2, b2):
    idx = inputs[:, :N_CAT].astype(jnp.int32)
    ff = jnp.arange(N_CAT, dtype=jnp.int32)[None, :]
    g = jnp.minimum(idx // VBLK, NBLK - 1)
    r = idx - jnp.minimum(g * VBLK, LAST_BASE)
    flat_idx = ((ff * NBLK + g) * 128 + r % 128) * 4 + r // 128
    idx3 = flat_idx.reshape(NW, NSLICE, SL)

    bb = jnp.arange(B, dtype=jnp.int32)[:, None]
    ii = jnp.arange(N_CAT, dtype=jnp.int32)[None, :]
    scat = ((bb // 8) * (XT_J * 32) + (ii // 4) * 32 + (bb % 8) * 4 + (ii % 4))
    scat3 = scat.reshape(NW, NSLICE, SL)

    t2 = jnp.swapaxes(tables, 1, 2)
    tab_lines = _relayout_tables(t2)
    flat_tables = tab_lines.reshape(TROWS, DIM)

    xflat = _sc_gather_scatter(flat_tables, idx3, scat3)
    x4 = xflat.reshape(XT_RB, XT_J, XT_S, XT_L)

    numeric = inputs[:, N_CAT:]
    return _mlp(x4, numeric, W1, b1.reshape(1, 128), W2, b2.reshape(1, 1))


# speedup vs baseline: 1.8417x; 1.0044x over previous
import functools

import jax
import jax.numpy as jnp
from jax import lax
from jax.experimental import pallas as pl
from jax.experimental.pallas import tpu as pltpu
from jax.experimental.pallas import tpu_sc as plsc

N_CAT = 26
N_NUM = 13
VOCAB = 100000
DIM = 32
B = 16384
ROWS = B * N_CAT
NC, NS = 2, 16
NW = NC * NS
ROWS_PER_W = ROWS // NW
SL = 128
NSLICE = ROWS_PER_W // SL
GRP = 4
GROUP_ROWS = GRP * SL
NGRP = NSLICE // GRP

VBLK = 512
NBLK = 196
LAST_BASE = VOCAB - VBLK
BPG = 196
NGSTEP = NBLK // BPG
LINES = N_CAT * NBLK * 128
TROWS = LINES * 4

XT_RB, XT_J, XT_S, XT_L = B // 8, 7, 8, 128
XCHUNKS = XT_RB * XT_J * XT_S * XT_L // DIM


def _relayout_tables(t2):
    SPAN = BPG * VBLK

    def body(t2_ref, out_ref):
        i = pl.program_id(1)

        @pl.when(i < NGSTEP - 1)
        def _():
            for k in range(BPG):
                x = t2_ref[0, :, pl.ds(k * VBLK, VBLK)]
                s = jnp.concatenate([x[:, 0:128], x[:, 128:256],
                                     x[:, 256:384], x[:, 384:512]], axis=0)
                out_ref[pl.ds(k * 128, 128), :] = s.T

        @pl.when(i == NGSTEP - 1)
        def _():
            for k in range(BPG):
                start = min((NGSTEP - 1) * SPAN + k * VBLK,
                            LAST_BASE) - (NGSTEP - 1) * SPAN
                x = t2_ref[0, :, pl.ds(start, VBLK)]
                s = jnp.concatenate([x[:, 0:128], x[:, 128:256],
                                     x[:, 256:384], x[:, 384:512]], axis=0)
                out_ref[pl.ds(k * 128, 128), :] = s.T

    return pl.pallas_call(
        body,
        grid=(N_CAT, NGSTEP),
        in_specs=[pl.BlockSpec((1, DIM, SPAN), lambda f, i: (f, 0, i))],
        out_specs=pl.BlockSpec((BPG * 128, 128), lambda f, i: (f * NGSTEP + i, 0)),
        out_shape=jax.ShapeDtypeStruct((LINES, 128), jnp.float32),
    )(t2)


def _sc_gather_scatter(flat_tables, idx3, scat3):
    mesh = plsc.VectorSubcoreMesh(core_axis_name="c", subcore_axis_name="s")

    @functools.partial(
        pl.kernel,
        mesh=mesh,
        compiler_params=pltpu.CompilerParams(use_tc_tiling_on_sc=False),
        out_type=jax.ShapeDtypeStruct((XCHUNKS, DIM), jnp.float32),
        scratch_types=[
            pltpu.VMEM((NSLICE, SL), jnp.int32),
            pltpu.VMEM((NSLICE, SL), jnp.int32),
            pltpu.VMEM((GROUP_ROWS, DIM), jnp.float32),
            pltpu.VMEM((GROUP_ROWS, DIM), jnp.float32),
            pltpu.SemaphoreType.DMA,
            pltpu.SemaphoreType.DMA,
            pltpu.SemaphoreType.DMA,
            pltpu.SemaphoreType.DMA,
        ],
    )
    def k(tab_hbm, idx_hbm, scat_hbm, out_hbm, idx_v, scat_v,
          buf0, buf1, gsem0, gsem1, wsem0, wsem1):
        wid = lax.axis_index("s") * NC + lax.axis_index("c")
        pltpu.sync_copy(idx_hbm.at[wid], idx_v)
        pltpu.sync_copy(scat_hbm.at[wid], scat_v)

        def fire_gather(g, buf, sem):
            for j in range(GRP):
                pltpu.async_copy(
                    tab_hbm.at[idx_v.at[g * GRP + j]],
                    buf.at[pl.ds(j * SL, SL)], sem)

        def drain(buf, sem, n=GRP):
            for j in range(n):
                pltpu.make_async_copy(
                    tab_hbm.at[idx_v.at[0]], buf.at[pl.ds(j * SL, SL)], sem
                ).wait()

        def fire_scatter(g, buf, sem):
            for j in range(GRP):
                pltpu.async_copy(
                    buf.at[pl.ds(j * SL, SL)],
                    out_hbm.at[scat_v.at[g * GRP + j]], sem)

        fire_gather(0, buf0, gsem0)

        def body(g, carry):
            def phase(buf, gsem, wsem, obuf, ogsem):
                drain(buf, gsem)
                @pl.when(g + 1 < NGRP)
                def _():
                    fire_gather(g + 1, obuf, ogsem)
                fire_scatter(g, buf, wsem)
                drain(buf, wsem)

            @pl.when(g % 2 == 0)
            def _():
                phase(buf0, gsem0, wsem0, buf1, gsem1)

            @pl.when(g % 2 == 1)
            def _():
                phase(buf1, gsem1, wsem1, buf0, gsem0)

            return carry

        lax.fori_loop(0, NGRP, body, 0)

    return k(flat_tables, idx3, scat3)


def _mlp(x4, numeric, W1, b1r, W2, b2r):
    BK = 1024
    BKH = BK // 8

    def body(x4_ref, num_ref, w1_ref, b1_ref, w2_ref, b2_ref, out_ref):
        parts = [x4_ref[:, j, :, :].reshape(BK, 128) for j in range(XT_J - 1)]
        parts.append(x4_ref[:, XT_J - 1, :, :].reshape(BK, 128)[:, :64])
        parts.append(num_ref[...])
        x = jnp.concatenate(parts, axis=1)
        h = jnp.dot(x, w1_ref[...], preferred_element_type=jnp.float32)
        h = jnp.maximum(h + b1_ref[...], 0.0)
        o = jnp.dot(h, w2_ref[...], preferred_element_type=jnp.float32) + b2_ref[0, 0]
        out_ref[...] = 1.0 / (1.0 + jnp.exp(-o))

    return pl.pallas_call(
        body,
        grid=(B // BK,),
        in_specs=[
            pl.BlockSpec((BKH, XT_J, XT_S, XT_L), lambda i: (i, 0, 0, 0)),
            pl.BlockSpec((BK, N_NUM), lambda i: (i, 0)),
            pl.BlockSpec((N_CAT * DIM + N_NUM, 128), lambda i: (0, 0)),
            pl.BlockSpec((1, 128), lambda i: (0, 0)),
            pl.BlockSpec((128, 1), lambda i: (0, 0)),
            pl.BlockSpec((1, 1), lambda i: (0, 0)),
        ],
        out_specs=pl.BlockSpec((BK, 1), lambda i: (i, 0)),
        out_shape=jax.ShapeDtypeStruct((B, 1), jnp.float32),
    )(x4, numeric, W1, b1r, W2, b2r)


def kernel(inputs, tables, W1, b1, W2, b2):
    idx = inputs[:, :N_CAT].astype(jnp.int32)
    ff = jnp.arange(N_CAT, dtype=jnp.int32)[None, :]
    g = jnp.minimum(idx // VBLK, NBLK - 1)
    r = idx - jnp.minimum(g * VBLK, LAST_BASE)
    flat_idx = ((ff * NBLK + g) * 128 + r % 128) * 4 + r // 128
    idx3 = flat_idx.reshape(NW, NSLICE, SL)

    bb = jnp.arange(B, dtype=jnp.int32)[:, None]
    ii = jnp.arange(N_CAT, dtype=jnp.int32)[None, :]
    scat = ((bb // 8) * (XT_J * 32) + (ii // 4) * 32 + (bb % 8) * 4 + (ii % 4))
    scat3 = scat.reshape(NW, NSLICE, SL)

    t2 = jnp.swapaxes(tables, 1, 2)
    tab_lines = _relayout_tables(t2)
    flat_tables = tab_lines.reshape(TROWS, DIM)

    xflat = _sc_gather_scatter(flat_tables, idx3, scat3)
    x4 = xflat.reshape(XT_RB, XT_J, XT_S, XT_L)

    numeric = inputs[:, N_CAT:]
    return _mlp(x4, numeric, W1, b1.reshape(1, 128), W2, b2.reshape(1, 1))
